# baseline (speedup 1.0000x reference)
import functools

import jax
import jax.numpy as jnp
from jax import lax
from jax.experimental import pallas as pl
from jax.experimental.pallas import tpu as pltpu
from jax.experimental.pallas import tpu_sc as plsc

N_SUB = 16
N_CW = 256
Q = 1024
K = 4096
LANES = 16
NUM_WORKERS = 32
Q_PER_W = Q // NUM_WORKERS
QBLK = 8
N_QBLK = Q_PER_W // QBLK
K_HALF = K // 2
CHUNKS_PER_HALF = K_HALF // LANES


def _sc_kernel(table_hbm, qidx_hbm, kidxt_hbm, out_hbm,
               kidx_v, qidx_v, g_v, out_v, gsem):
    wid = lax.axis_index("s") * 2 + lax.axis_index("c")

    pltpu.sync_copy(kidxt_hbm, kidx_v)
    pltpu.sync_copy(qidx_hbm.at[pl.ds(wid * N_QBLK, N_QBLK)], qidx_v)

    def per_qblock(qb, _):
        pltpu.async_copy(table_hbm.at[qidx_v.at[qb]], g_v, gsem).wait()

        for h in range(2):
            def per_chunk(c, _):
                accs = [None] * QBLK
                for i in range(N_SUB):
                    kv = kidx_v[i, pl.ds(h * K_HALF + c * LANES, LANES)]
                    for b in range(QBLK):
                        g = plsc.load_gather(g_v.at[b * N_SUB + i], [kv])
                        accs[b] = g if i == 0 else accs[b] + g
                for b in range(QBLK):
                    out_v[b, pl.ds(c * LANES, LANES)] = accs[b]
                return ()

            lax.fori_loop(0, CHUNKS_PER_HALF, per_chunk, ())
            pltpu.sync_copy(
                out_v,
                out_hbm.at[pl.ds((wid * N_QBLK + qb) * QBLK, QBLK),
                           pl.ds(h * K_HALF, K_HALF)])
        return ()

    lax.fori_loop(0, N_QBLK, per_qblock, ())


def kernel(q_code, k_code, table):
    table_flat = table.reshape(N_SUB * N_CW, N_CW)
    qidx = (q_code.astype(jnp.int32)
            + jnp.arange(N_SUB, dtype=jnp.int32)[None, :] * N_CW)
    qidx = qidx.reshape(NUM_WORKERS * N_QBLK, QBLK * N_SUB)
    kidxt = k_code.T.astype(jnp.int32)

    mesh = plsc.VectorSubcoreMesh(core_axis_name="c", subcore_axis_name="s")
    f = functools.partial(
        pl.kernel,
        mesh=mesh,
        compiler_params=pltpu.CompilerParams(use_tc_tiling_on_sc=False,
                                             needs_layout_passes=False),
        out_type=jax.ShapeDtypeStruct((Q, K), jnp.float32),
        scratch_types=[
            pltpu.VMEM((N_SUB, K), jnp.int32),
            pltpu.VMEM((N_QBLK, QBLK * N_SUB), jnp.int32),
            pltpu.VMEM((QBLK * N_SUB, N_CW), jnp.float32),
            pltpu.VMEM((QBLK, K_HALF), jnp.float32),
            pltpu.SemaphoreType.DMA,
        ],
    )(_sc_kernel)
    return f(table_flat, qidx, kidxt)


# speedup vs baseline: 2.4973x; 2.4973x over previous
import functools

import jax
import jax.numpy as jnp
from jax import lax
from jax.experimental import pallas as pl
from jax.experimental.pallas import tpu as pltpu
from jax.experimental.pallas import tpu_sc as plsc

N_SUB = 16
N_CW = 256
Q = 1024
K = 4096
LANES = 16
NUM_WORKERS = 32
Q_PER_W = Q // NUM_WORKERS
QBLK = 8
N_QBLK = Q_PER_W // QBLK

K_SC = 1024
K_TC = K - K_SC
KTILE = 512


def _make_sc(k_sc):
    n_flush = 2 if k_sc > 2048 else 1
    k_part = k_sc // n_flush
    chunks = k_part // LANES

    def body(table_hbm, qidx_hbm, kidxt_hbm, out_hbm,
             kidx_v, qidx_v, g_v, out_v, gsem):
        wid = lax.axis_index("s") * 2 + lax.axis_index("c")
        pltpu.sync_copy(kidxt_hbm, kidx_v)
        pltpu.sync_copy(qidx_hbm.at[pl.ds(wid * N_QBLK, N_QBLK)], qidx_v)

        def per_qblock(qb, _):
            pltpu.async_copy(table_hbm.at[qidx_v.at[qb]], g_v, gsem).wait()
            for h in range(n_flush):
                def per_chunk(c, _):
                    accs = [None] * QBLK
                    for i in range(N_SUB):
                        kv = kidx_v[i, pl.ds(h * k_part + c * LANES, LANES)]
                        for b in range(QBLK):
                            g = plsc.load_gather(
                                g_v,
                                [jnp.full((LANES,), b * N_SUB + i, jnp.int32),
                                 kv])
                            accs[b] = g if i == 0 else accs[b] + g
                    for b in range(QBLK):
                        out_v[b, pl.ds(c * LANES, LANES)] = accs[b]
                    return ()

                lax.fori_loop(0, chunks, per_chunk, ())
                pltpu.sync_copy(
                    out_v,
                    out_hbm.at[pl.ds((wid * N_QBLK + qb) * QBLK, QBLK),
                               pl.ds(h * k_part, k_part)])
            return ()

        lax.fori_loop(0, N_QBLK, per_qblock, ())

    mesh = plsc.VectorSubcoreMesh(core_axis_name="c", subcore_axis_name="s")
    return functools.partial(
        pl.kernel,
        mesh=mesh,
        compiler_params=pltpu.CompilerParams(use_tc_tiling_on_sc=False,
                                             needs_layout_passes=False),
        out_type=jax.ShapeDtypeStruct((Q, k_sc), jnp.float32),
        scratch_types=[
            pltpu.VMEM((N_SUB, k_sc), jnp.int32),
            pltpu.VMEM((N_QBLK, QBLK * N_SUB), jnp.int32),
            pltpu.VMEM((QBLK * N_SUB, N_CW), jnp.float32),
            pltpu.VMEM((QBLK, k_part), jnp.float32),
            pltpu.SemaphoreType.DMA,
        ],
    )(body)


def _sc_part(q_code, kc_slab, table):
    k_sc = kc_slab.shape[0]
    table_flat = table.reshape(N_SUB * N_CW, N_CW)
    qidx = (q_code.astype(jnp.int32)
            + jnp.arange(N_SUB, dtype=jnp.int32)[None, :] * N_CW)
    qidx = qidx.reshape(NUM_WORKERS * N_QBLK, QBLK * N_SUB)
    kidxt = kc_slab.T.astype(jnp.int32)
    return _make_sc(k_sc)(table_flat, qidx, kidxt)


def _tc_g_body(qc_ref, table_ref, g_ref):
    qc = qc_ref[...]
    iota = lax.broadcasted_iota(jnp.int32, (Q, N_CW), 1)
    for i in range(N_SUB):
        oh = (qc[:, i:i + 1] == iota).astype(jnp.bfloat16)
        t = table_ref[i].astype(jnp.bfloat16)
        g_ref[:, i * N_CW:(i + 1) * N_CW] = lax.dot_general(
            oh, t, (((1,), (0,)), ((), ())),
            preferred_element_type=jnp.float32).astype(jnp.bfloat16)


def _tc_out_body(kc_ref, g_ref, out_ref):
    acc = jnp.zeros((Q, KTILE), jnp.float32)
    iota = lax.broadcasted_iota(jnp.int32, (N_CW, KTILE), 0)
    for i in range(N_SUB):
        oh = (kc_ref[i] == iota).astype(jnp.bfloat16)
        gi = g_ref[:, i * N_CW:(i + 1) * N_CW]
        acc = acc + lax.dot_general(
            gi, oh, (((1,), (0,)), ((), ())),
            preferred_element_type=jnp.float32)
    out_ref[...] = acc


def _tc_part(q_code, kc_slab, table):
    kt = kc_slab.shape[0]
    g = pl.pallas_call(
        _tc_g_body,
        out_shape=jax.ShapeDtypeStruct((Q, N_SUB * N_CW), jnp.bfloat16),
    )(q_code.astype(jnp.int32), table)

    kc3 = kc_slab.T.reshape(N_SUB, 1, kt).astype(jnp.int32)
    return pl.pallas_call(
        _tc_out_body,
        grid=(kt // KTILE,),
        in_specs=[
            pl.BlockSpec((N_SUB, 1, KTILE), lambda j: (0, 0, j)),
            pl.BlockSpec((Q, N_SUB * N_CW), lambda j: (0, 0)),
        ],
        out_specs=pl.BlockSpec((Q, KTILE), lambda j: (0, j)),
        out_shape=jax.ShapeDtypeStruct((Q, kt), jnp.float32),
    )(kc3, g)


def kernel(q_code, k_code, table):
    out_tc = _tc_part(q_code, k_code[:K_TC], table)
    out_sc = _sc_part(q_code, k_code[K_TC:], table)
    return jnp.concatenate([out_tc, out_sc], axis=1)


# speedup vs baseline: 2.6409x; 1.0575x over previous
import functools

import jax
import jax.numpy as jnp
from jax import lax
from jax.experimental import pallas as pl
from jax.experimental.pallas import tpu as pltpu
from jax.experimental.pallas import tpu_sc as plsc

N_SUB = 16
N_CW = 256
Q = 1024
K = 4096
LANES = 16
NUM_WORKERS = 32

Q_SC = 224
Q_TC = Q - Q_SC
QBLK = Q_SC // NUM_WORKERS
KTILE = 512
N_FLUSH = 2
K_PART = K // N_FLUSH
CHUNKS = K_PART // LANES


def _sc_body(table_hbm, qidx_hbm, kidxt_hbm, out_hbm,
             kidx_v, qidx_v, g_v, out_v, gsem):
    wid = lax.axis_index("s") * 2 + lax.axis_index("c")
    pltpu.sync_copy(kidxt_hbm, kidx_v)
    pltpu.sync_copy(qidx_hbm.at[pl.ds(wid, 1)], qidx_v)
    pltpu.async_copy(table_hbm.at[qidx_v.at[0]], g_v, gsem).wait()

    for h in range(N_FLUSH):
        def per_chunk(c, _):
            accs = [None] * QBLK
            for i in range(N_SUB):
                kv = kidx_v[i, pl.ds(h * K_PART + c * LANES, LANES)]
                for b in range(QBLK):
                    g = plsc.load_gather(
                        g_v, [jnp.full((LANES,), b * N_SUB + i, jnp.int32),
                              kv])
                    accs[b] = g if i == 0 else accs[b] + g
            for b in range(QBLK):
                out_v[b, pl.ds(c * LANES, LANES)] = accs[b]
            return ()

        lax.fori_loop(0, CHUNKS, per_chunk, ())
        pltpu.sync_copy(out_v,
                        out_hbm.at[pl.ds(wid * QBLK, QBLK),
                                   pl.ds(h * K_PART, K_PART)])


def _sc_part(qc_rows, k_code, table):
    table_flat = table.reshape(N_SUB * N_CW, N_CW)
    qidx = (qc_rows.astype(jnp.int32)
            + jnp.arange(N_SUB, dtype=jnp.int32)[None, :] * N_CW)
    qidx = qidx.reshape(NUM_WORKERS, QBLK * N_SUB)
    kidxt = k_code.T.astype(jnp.int32)

    mesh = plsc.VectorSubcoreMesh(core_axis_name="c", subcore_axis_name="s")
    f = functools.partial(
        pl.kernel,
        mesh=mesh,
        compiler_params=pltpu.CompilerParams(use_tc_tiling_on_sc=False,
                                             needs_layout_passes=False),
        out_type=jax.ShapeDtypeStruct((Q_SC, K), jnp.float32),
        scratch_types=[
            pltpu.VMEM((N_SUB, K), jnp.int32),
            pltpu.VMEM((1, QBLK * N_SUB), jnp.int32),
            pltpu.VMEM((QBLK * N_SUB, N_CW), jnp.float32),
            pltpu.VMEM((QBLK, K_PART), jnp.float32),
            pltpu.SemaphoreType.DMA,
        ],
    )(_sc_body)
    return f(table_flat, qidx, kidxt)


def _tc_body(qc_ref, table_ref, kc_ref, out_ref, g_scr):
    @pl.when(pl.program_id(0) == 0)
    def _build_g():
        qc = qc_ref[...]
        iota_q = lax.broadcasted_iota(jnp.int32, (Q_TC, N_CW), 1)
        for i in range(N_SUB):
            oh = (qc[:, i:i + 1] == iota_q).astype(jnp.bfloat16)
            g_scr[:, i * N_CW:(i + 1) * N_CW] = lax.dot_general(
                oh, table_ref[i], (((1,), (0,)), ((), ())),
                preferred_element_type=jnp.float32).astype(jnp.bfloat16)

    acc = jnp.zeros((Q_TC, KTILE), jnp.float32)
    iota_k = lax.broadcasted_iota(jnp.int32, (N_CW, KTILE), 0)
    for i in range(N_SUB):
        oh = (kc_ref[i] == iota_k).astype(jnp.bfloat16)
        acc = acc + lax.dot_general(
            g_scr[:, i * N_CW:(i + 1) * N_CW], oh, (((1,), (0,)), ((), ())),
            preferred_element_type=jnp.float32)
    out_ref[...] = acc


def _tc_part(qc_rows, k_code, table_bf16):
    kc3 = k_code.T.reshape(N_SUB, 1, K).astype(jnp.int32)
    return pl.pallas_call(
        _tc_body,
        grid=(K // KTILE,),
        in_specs=[
            pl.BlockSpec((Q_TC, N_SUB), lambda j: (0, 0)),
            pl.BlockSpec((N_SUB, N_CW, N_CW), lambda j: (0, 0, 0)),
            pl.BlockSpec((N_SUB, 1, KTILE), lambda j: (0, 0, j)),
        ],
        out_specs=pl.BlockSpec((Q_TC, KTILE), lambda j: (0, j)),
        out_shape=jax.ShapeDtypeStruct((Q_TC, K), jnp.float32),
        scratch_shapes=[pltpu.VMEM((Q_TC, N_SUB * N_CW), jnp.bfloat16)],
    )(qc_rows.astype(jnp.int32), table_bf16, kc3)


def kernel(q_code, k_code, table):
    out_tc = _tc_part(q_code[:Q_TC], k_code, table.astype(jnp.bfloat16))
    out_sc = _sc_part(q_code[Q_TC:], k_code, table)
    return jnp.concatenate([out_tc, out_sc], axis=0)


# speedup vs baseline: 2.7850x; 1.0546x over previous
import functools

import jax
import jax.numpy as jnp
from jax import lax
from jax.experimental import pallas as pl
from jax.experimental.pallas import tpu as pltpu
from jax.experimental.pallas import tpu_sc as plsc

N_SUB = 16
N_CW = 256
Q = 1024
K = 4096
LANES = 16
NUM_WORKERS = 32

Q_SC = 160
Q_TC = Q - Q_SC
QBLK = Q_SC // NUM_WORKERS
KTILE = 512
N_FLUSH = 2
K_PART = K // N_FLUSH
CHUNKS = K_PART // LANES


def _sc_body(table_hbm, qidx_hbm, kidxt_hbm, out_hbm,
             kidx_v, qidx_v, g_v, out_v, gsem):
    wid = lax.axis_index("s") * 2 + lax.axis_index("c")
    pltpu.sync_copy(kidxt_hbm, kidx_v)
    pltpu.sync_copy(qidx_hbm.at[pl.ds(wid, 1)], qidx_v)
    pltpu.async_copy(table_hbm.at[qidx_v.at[0]], g_v, gsem).wait()

    for h in range(N_FLUSH):
        def per_chunk(c, _):
            accs = [None] * QBLK
            for i in range(N_SUB):
                kv = kidx_v[i, pl.ds(h * K_PART + c * LANES, LANES)]
                for b in range(QBLK):
                    g = plsc.load_gather(
                        g_v, [jnp.full((LANES,), b * N_SUB + i, jnp.int32),
                              kv])
                    accs[b] = g if i == 0 else accs[b] + g
            for b in range(QBLK):
                out_v[b, pl.ds(c * LANES, LANES)] = accs[b]
            return ()

        lax.fori_loop(0, CHUNKS, per_chunk, ())
        pltpu.sync_copy(out_v,
                        out_hbm.at[pl.ds(wid * QBLK, QBLK),
                                   pl.ds(h * K_PART, K_PART)])


def _sc_part(qc_rows, k_code, table):
    table_flat = table.reshape(N_SUB * N_CW, N_CW)
    qidx = (qc_rows.astype(jnp.int32)
            + jnp.arange(N_SUB, dtype=jnp.int32)[None, :] * N_CW)
    qidx = qidx.reshape(NUM_WORKERS, QBLK * N_SUB)
    kidxt = k_code.T.astype(jnp.int32)

    mesh = plsc.VectorSubcoreMesh(core_axis_name="c", subcore_axis_name="s")
    f = functools.partial(
        pl.kernel,
        mesh=mesh,
        compiler_params=pltpu.CompilerParams(use_tc_tiling_on_sc=False,
                                             needs_layout_passes=False),
        out_type=jax.ShapeDtypeStruct((Q_SC, K), jnp.float32),
        scratch_types=[
            pltpu.VMEM((N_SUB, K), jnp.int32),
            pltpu.VMEM((1, QBLK * N_SUB), jnp.int32),
            pltpu.VMEM((QBLK * N_SUB, N_CW), jnp.float32),
            pltpu.VMEM((QBLK, K_PART), jnp.float32),
            pltpu.SemaphoreType.DMA,
        ],
    )(_sc_body)
    return f(table_flat, qidx, kidxt)


def _tc_body(qc_ref, table_ref, kc_ref, out_ref, g_scr):
    @pl.when(pl.program_id(0) == 0)
    def _build_g():
        qc = qc_ref[...]
        iota_q = lax.broadcasted_iota(jnp.int32, (Q_TC, N_CW), 1)
        for i in range(N_SUB):
            oh = (qc[:, i:i + 1] == iota_q).astype(jnp.bfloat16)
            g_scr[:, i * N_CW:(i + 1) * N_CW] = lax.dot_general(
                oh, table_ref[i], (((1,), (0,)), ((), ())),
                preferred_element_type=jnp.float32).astype(jnp.bfloat16)

    acc = jnp.zeros((Q_TC, KTILE), jnp.float32)
    iota_k = lax.broadcasted_iota(jnp.int32, (N_CW, KTILE), 0)
    for i in range(N_SUB):
        oh = (kc_ref[i] == iota_k).astype(jnp.bfloat16)
        acc = acc + lax.dot_general(
            g_scr[:, i * N_CW:(i + 1) * N_CW], oh, (((1,), (0,)), ((), ())),
            preferred_element_type=jnp.float32)
    out_ref[...] = acc


def _tc_part(qc_rows, k_code, table_bf16):
    kc3 = k_code.T.reshape(N_SUB, 1, K).astype(jnp.int32)
    return pl.pallas_call(
        _tc_body,
        grid=(K // KTILE,),
        in_specs=[
            pl.BlockSpec((Q_TC, N_SUB), lambda j: (0, 0)),
            pl.BlockSpec((N_SUB, N_CW, N_CW), lambda j: (0, 0, 0)),
            pl.BlockSpec((N_SUB, 1, KTILE), lambda j: (0, 0, j)),
        ],
        out_specs=pl.BlockSpec((Q_TC, KTILE), lambda j: (0, j)),
        out_shape=jax.ShapeDtypeStruct((Q_TC, K), jnp.float32),
        scratch_shapes=[pltpu.VMEM((Q_TC, N_SUB * N_CW), jnp.bfloat16)],
    )(qc_rows.astype(jnp.int32), table_bf16, kc3)


def _asm_body(tc_ref, sc_ref, out_ref):
    out_ref[:Q_TC, :] = tc_ref[...]
    out_ref[Q_TC:, :] = sc_ref[...]


def _assemble(out_tc, out_sc):
    return pl.pallas_call(
        _asm_body,
        grid=(K // KTILE,),
        in_specs=[
            pl.BlockSpec((Q_TC, KTILE), lambda j: (0, j)),
            pl.BlockSpec((Q_SC, KTILE), lambda j: (0, j)),
        ],
        out_specs=pl.BlockSpec((Q, KTILE), lambda j: (0, j)),
        out_shape=jax.ShapeDtypeStruct((Q, K), jnp.float32),
    )(out_tc, out_sc)


def kernel(q_code, k_code, table):
    out_tc = _tc_part(q_code[:Q_TC], k_code, table.astype(jnp.bfloat16))
    out_sc = _sc_part(q_code[Q_TC:], k_code, table)
    return _assemble(out_tc, out_sc)
